# Initial kernel scaffold; baseline (speedup 1.0000x reference)
import jax
import jax.numpy as jnp
from jax.experimental import pallas as pl


def kernel(x, subg_nodes, local_src, local_dst, local_attr, X_val, batch, x_table, ea_table, tf_table, lin0_W, lin0_b, lin1_W, lin1_b, conv_W1, conv_b1, conv_W2, conv_b2, pred_W, pred_b):
    raise NotImplementedError("write your pallas kernel here")


# speedup vs baseline: 3.8250x; 3.8250x over previous
import functools

import jax
import jax.numpy as jnp
from jax import lax
from jax.experimental import pallas as pl
from jax.experimental.pallas import tpu as pltpu
from jax.experimental.pallas import tpu_sc as plsc

N = 10000
K = 16
L = 32
D = 128
NLAYER = 3
NG = 64

BP = 2000
NBP = N // BP

SC_NC = 2
SC_NS = 16
NW = SC_NC * SC_NS
CH = 128
CPW = 40
BPAD = NW * CPW * CH

BN = 400
NB = N // BN
BSUB = 8
NSUB = BN // BSUB
SPB = BSUB * K
EPB = BSUB * L
BNK = BN * K
BNL = BN * L


def _prep_body(x_ref, xtab_ref, w0_ref, b0_ref, w1_ref, b1_ref, t0_ref, t1_ref):
    f32 = jnp.float32
    oh = (x_ref[...] == lax.broadcasted_iota(jnp.int32, (BP, 32), 1)).astype(f32)
    xe = jnp.dot(oh, xtab_ref[...], preferred_element_type=f32)
    t0_ref[...] = jnp.dot(xe, w0_ref[...], preferred_element_type=f32) + b0_ref[...]
    t1_ref[...] = jnp.dot(xe, w1_ref[...], preferred_element_type=f32) + b1_ref[...]


def _sc_gather_body(t1_hbm, idx_hbm, out_hbm, idx_v, rows_v, sem):
    wid = lax.axis_index("s") * SC_NC + lax.axis_index("c")
    pltpu.sync_copy(idx_hbm.at[pl.ds(wid * CPW, CPW)], idx_v)

    def chunk(j, carry):
        pltpu.async_copy(t1_hbm.at[idx_v.at[j]], rows_v, sem).wait()
        pltpu.sync_copy(rows_v, out_hbm.at[pl.ds(wid * CPW * CH + j * CH, CH)])
        return carry

    lax.fori_loop(0, CPW, chunk, 0)


@functools.cache
def _sc_gather():
    return pl.kernel(
        _sc_gather_body,
        out_type=jax.ShapeDtypeStruct((BPAD, D), jnp.float32),
        mesh=plsc.VectorSubcoreMesh(core_axis_name="c", subcore_axis_name="s"),
        scratch_types=[
            pltpu.VMEM((CPW, CH), jnp.int32),
            pltpu.VMEM((CH, D), jnp.float32),
            pltpu.SemaphoreType.DMA,
        ],
    )


def _main_body(t0_ref, g_ref, xval_ref, gsrc_ref, gdst_ref, attr_ref, batch_ref,
               eatab_ref, tftab_ref, cw1_ref, cb1_ref, cw2_ref, cb2_ref,
               pw_ref, pb_ref, out_ref, x_sc, ea_sc, agg_sc, hg_sc):
    f32 = jnp.float32
    b = pl.program_id(0)

    oha = (attr_ref[...] == lax.broadcasted_iota(jnp.int32, (BNL, 16), 1)).astype(f32)
    ea_sc[...] = jnp.dot(oha, eatab_ref[...], preferred_element_type=f32)

    oht = (xval_ref[...] == lax.broadcasted_iota(jnp.int32, (BNK, 16), 1)).astype(f32)
    xt = jnp.dot(oht, tftab_ref[...], preferred_element_type=f32)
    t0b = jnp.broadcast_to(t0_ref[...][:, None, :], (BN, K, D)).reshape(BNK, D)
    x_sc[...] = t0b * g_ref[...] * xt

    for l in range(NLAYER):
        def sub(s, carry):
            gs = gsrc_ref[pl.ds(s * EPB, EPB), :]
            ohs = (gs == lax.broadcasted_iota(jnp.int32, (EPB, SPB), 1)).astype(f32)
            xs = x_sc[pl.ds(s * SPB, SPB), :]
            srcf = jnp.dot(ohs, xs, preferred_element_type=f32)
            msg = srcf * ea_sc[pl.ds(s * EPB, EPB), :]
            gd = gdst_ref[:, pl.ds(s, 1), :].reshape(1, EPB)
            ohd = (lax.broadcasted_iota(jnp.int32, (SPB, EPB), 0) == gd).astype(f32)
            agg_sc[pl.ds(s * SPB, SPB), :] = jnp.dot(ohd, msg, preferred_element_type=f32)
            return carry

        lax.fori_loop(0, NSUB, sub, 0, unroll=2)

        h = jnp.maximum(
            jnp.dot(agg_sc[...], cw1_ref[l], preferred_element_type=f32)
            + cb1_ref[l:l + 1, :], 0.0)
        x_sc[...] = (x_sc[...]
                     + jnp.dot(h, cw2_ref[l], preferred_element_type=f32)
                     + cb2_ref[l:l + 1, :])

    x3 = x_sc[...].reshape(BN, K, D)
    xnode = x3[:, 0, :]
    for k in range(1, K):
        xnode = jnp.maximum(xnode, x3[:, k, :])

    bt = batch_ref[...].reshape(1, BN)
    ohb = (lax.broadcasted_iota(jnp.int32, (NG, BN), 0) == bt).astype(f32)
    contrib = jnp.dot(ohb, xnode, preferred_element_type=f32)

    @pl.when(b == 0)
    def _():
        hg_sc[...] = contrib

    @pl.when(b > 0)
    def _():
        hg_sc[...] = hg_sc[...] + contrib

    @pl.when(b == NB - 1)
    def _():
        out_ref[...] = (jnp.dot(hg_sc[...], pw_ref[...], preferred_element_type=f32)
                        + pb_ref[...])


def _full(shape):
    return pl.BlockSpec(shape, lambda i: (0,) * len(shape))


_prep_call = pl.pallas_call(
    _prep_body,
    grid=(NBP,),
    in_specs=[
        pl.BlockSpec((BP, 1), lambda i: (i, 0)),
        _full((32, D)), _full((D, D)), _full((1, D)), _full((D, D)), _full((1, D)),
    ],
    out_specs=[
        pl.BlockSpec((BP, D), lambda i: (i, 0)),
        pl.BlockSpec((BP, D), lambda i: (i, 0)),
    ],
    out_shape=[
        jax.ShapeDtypeStruct((N, D), jnp.float32),
        jax.ShapeDtypeStruct((N, D), jnp.float32),
    ],
)

_main_call = pl.pallas_call(
    _main_body,
    grid=(NB,),
    in_specs=[
        pl.BlockSpec((BN, D), lambda i: (i, 0)),
        pl.BlockSpec((BNK, D), lambda i: (i, 0)),
        pl.BlockSpec((BNK, 1), lambda i: (i, 0)),
        pl.BlockSpec((BNL, 1), lambda i: (i, 0)),
        pl.BlockSpec((1, NSUB, EPB), lambda i: (i, 0, 0)),
        pl.BlockSpec((BNL, 1), lambda i: (i, 0)),
        pl.BlockSpec((1, 1, BN), lambda i: (i, 0, 0)),
        _full((16, D)), _full((16, D)),
        _full((NLAYER, D, D)), _full((NLAYER, D)),
        _full((NLAYER, D, D)), _full((NLAYER, D)),
        _full((D, 1)), _full((1, 1)),
    ],
    out_specs=pl.BlockSpec((NG, 1), lambda i: (0, 0)),
    out_shape=jax.ShapeDtypeStruct((NG, 1), jnp.float32),
    scratch_shapes=[
        pltpu.VMEM((BNK, D), jnp.float32),
        pltpu.VMEM((BNL, D), jnp.float32),
        pltpu.VMEM((BNK, D), jnp.float32),
        pltpu.VMEM((NG, D), jnp.float32),
    ],
    compiler_params=pltpu.CompilerParams(
        dimension_semantics=("arbitrary",),
    ),
)


def kernel(x, subg_nodes, local_src, local_dst, local_attr, X_val, batch,
           x_table, ea_table, tf_table, lin0_W, lin0_b, lin1_W, lin1_b,
           conv_W1, conv_b1, conv_W2, conv_b2, pred_W, pred_b):
    i32 = jnp.int32
    x2 = x.astype(i32).reshape(N, 1)
    t0, t1 = _prep_call(x2, x_table, lin0_W, lin0_b.reshape(1, D),
                        lin1_W, lin1_b.reshape(1, D))

    idx = subg_nodes.astype(i32).reshape(N * K)
    idx = jnp.concatenate([idx, jnp.zeros((BPAD - N * K,), i32)]).reshape(BPAD // CH, CH)
    g = _sc_gather()(t1, idx)[:N * K]

    roff = (jnp.arange(N, dtype=i32)[:, None] % BSUB) * K
    gsrc = (roff + local_src.astype(i32)).reshape(N * L, 1)
    gdst = (roff + local_dst.astype(i32)).reshape(NB, NSUB, EPB)
    out = _main_call(
        t0, g,
        X_val.astype(i32).reshape(N * K, 1),
        gsrc, gdst,
        local_attr.astype(i32).reshape(N * L, 1),
        batch.astype(i32).reshape(NB, 1, BN),
        ea_table, tf_table, conv_W1, conv_b1, conv_W2, conv_b2,
        pred_W, pred_b.reshape(1, 1),
    )
    return out
